# baseline (speedup 1.0000x reference)
import functools
import math

import jax
import jax.numpy as jnp
from jax import lax
from jax.experimental import pallas as pl
from jax.experimental.pallas import tpu as pltpu
from jax.experimental.pallas import tpu_sc as plsc

_LANE = 16
_TL = 128


def _mesh():
    return plsc.VectorSubcoreMesh(core_axis_name="c", subcore_axis_name="s")


def _wid(nc):
    return lax.axis_index("s") * nc + lax.axis_index("c")


@functools.lru_cache(maxsize=None)
def _make_retile(V, D, S):
    info = plsc.get_sparse_core_info()
    nc, ns = info.num_cores, info.num_subcores
    nw = nc * ns
    n_blk = V // _TL
    rem = V - n_blk * _TL
    n_outer = (n_blk + nw - 1) // nw
    rpb = _TL * D // _TL
    assert D % _LANE == 0 and (rem * D) % _TL == 0

    @functools.partial(
        pl.kernel,
        mesh=_mesh(),
        compiler_params=pltpu.CompilerParams(
            use_tc_tiling_on_sc=True, needs_layout_passes=False
        ),
        out_type=jax.ShapeDtypeStruct((V * D // _TL, _TL), jnp.float32),
        scratch_types=[
            pltpu.VMEM((S, D, _TL), jnp.float32),
            pltpu.VMEM((S, rpb, _TL), jnp.float32),
            pltpu.VMEM((D, rem if rem else 1), jnp.float32),
            pltpu.VMEM((max(rem * D // _TL, 1), _TL), jnp.float32),
        ]
        + [pltpu.SemaphoreType.DMA] * (2 * S),
    )
    def retile(tt_hbm, out_hbm, tin_v, tout_v, rin_v, rout_v, *sems):
        sem_g = sems[:S]
        sem_w = sems[S:]
        w = _wid(nc)
        iota = lax.iota(jnp.int32, _LANE)
        rows01 = [iota + j * _LANE for j in range(D // _LANE)]
        vper = _TL // D

        def fire_read(b, c):
            pltpu.async_copy(
                tt_hbm.at[:, pl.ds(c * _TL, _TL)], tin_v.at[b], sem_g[b]
            )

        def drain_read(b):
            pltpu.make_async_copy(
                tt_hbm.at[:, pl.ds(0, _TL)], tin_v.at[b], sem_g[b]
            ).wait()

        def wait_write(b):
            pltpu.make_async_copy(
                tout_v.at[b], out_hbm.at[pl.ds(0, rpb)], sem_w[b]
            ).wait()

        def transpose_blk(src, dst, nv):
            for r in range(nv * D // _TL):
                for k in range(_TL // _LANE):
                    vp = (r * _TL + k * _LANE) // D
                    col = jnp.full((_LANE,), vp, jnp.int32)
                    row = rows01[(k * _LANE % D) // _LANE]
                    vals = plsc.load_gather(src, [row, col])
                    dst[r, pl.ds(k * _LANE, _LANE)] = vals

        for b in range(S):
            c = b * nw + w

            @pl.when(c < n_blk)
            def _():
                fire_read(b, c)

        def outer(g, carry):
            for b in range(S):
                c = (g * S + b) * nw + w

                @pl.when(c < n_blk)
                def _():
                    drain_read(b)
                    transpose_blk(tin_v.at[b], tout_v.at[b], _TL)
                    pltpu.async_copy(
                        tout_v.at[b],
                        out_hbm.at[pl.ds(c * rpb, rpb)],
                        sem_w[b],
                    )
                    t = c + S * nw

                    @pl.when(t < n_blk)
                    def _():
                        wait_write(b)
                        fire_read(b, t)
            return carry

        lax.fori_loop(0, n_outer // S + 1, outer, 0)
        for b in range(S):
            c0 = b * nw + w

            @pl.when(c0 < n_blk)
            def _():
                wait_write(b)

        if rem:
            @pl.when(w == nw - 1)
            def _():
                pltpu.sync_copy(tt_hbm.at[:, pl.ds(n_blk * _TL, rem)], rin_v)
                transpose_blk(rin_v, rout_v, rem)
                pltpu.sync_copy(
                    rout_v,
                    out_hbm.at[pl.ds(n_blk * rpb, rem * D // _TL)],
                )

    return retile


@functools.lru_cache(maxsize=None)
def _make_gather(BT, F, V, D, S):
    info = plsc.get_sparse_core_info()
    nc, ns = info.num_cores, info.num_subcores
    nw = nc * ns
    bblk = BT // _TL
    n_chunk = F * bblk
    per_w = n_chunk // nw
    scale = math.sqrt(D)
    assert BT % _TL == 0 and n_chunk % nw == 0 and per_w % S == 0

    @functools.partial(
        pl.kernel,
        mesh=_mesh(),
        compiler_params=pltpu.CompilerParams(use_tc_tiling_on_sc=False, needs_layout_passes=False),
        out_type=jax.ShapeDtypeStruct((F, D // 8, bblk, 8, _TL), jnp.float32),
        scratch_types=[
            pltpu.VMEM((S, _TL), jnp.int32),
            pltpu.VMEM((S, _TL, D), jnp.float32),
            pltpu.VMEM((S, D // 8, 8, _TL), jnp.float32),
        ]
        + [pltpu.SemaphoreType.DMA] * (2 * S),
    )
    def gatherk(x3_hbm, tab_hbm, out_hbm, ix_v, rows_v, outt_v, *sems):
        sem_g = sems[:S]
        sem_w = sems[S:]
        w = _wid(nc)
        cbase = w * per_w
        iota = lax.iota(jnp.int32, _LANE)
        lanes = [iota + i * _LANE for i in range(_TL // _LANE)]

        def fetch_and_fire(b, c):
            f = c // bblk
            bb = c % bblk
            pltpu.sync_copy(x3_hbm.at[f, bb], ix_v.at[b])
            pltpu.async_copy(tab_hbm.at[ix_v.at[b]], rows_v.at[b], sem_g[b])

        def drain_gather(b):
            pltpu.make_async_copy(
                tab_hbm.at[pl.ds(0, _TL)], rows_v.at[b], sem_g[b]
            ).wait()

        def wait_write(b):
            pltpu.make_async_copy(
                outt_v.at[b], out_hbm.at[0, :, 0], sem_w[b]
            ).wait()

        def extract(b):
            for i in range(_TL // _LANE):
                for d in range(D):
                    col = jnp.full((_LANE,), d, jnp.int32)
                    vals = plsc.load_gather(rows_v.at[b], [lanes[i], col])
                    outt_v[b, d // 8, d % 8, pl.ds(i * _LANE, _LANE)] = (
                        vals * scale
                    )

        for b in range(S):
            fetch_and_fire(b, cbase + b)

        def outer(g, carry):
            for b in range(S):
                c = cbase + g * S + b
                drain_gather(b)
                extract(b)
                f = c // bblk
                bb = c % bblk
                pltpu.async_copy(
                    outt_v.at[b], out_hbm.at[f, :, bb], sem_w[b]
                )
                bp = (b - 1) % S
                t = c + S - 1
                ok = jnp.logical_and(g * S + b >= 1,
                                     t <= cbase + per_w - 1)

                @pl.when(ok)
                def _():
                    wait_write(bp)
                    fetch_and_fire(bp, t)
            return carry

        lax.fori_loop(0, per_w // S, outer, 0)
        for b in range(S):
            wait_write(b)

    return gatherk


def kernel(x, table):
    bt, f = x.shape
    v, d = table.shape
    tab128 = _make_retile(v, d, 3)(table.T)
    tab_rm = tab128.reshape(v, d)
    x3 = x.T.astype(jnp.int32).reshape(f, bt // _TL, _TL)
    out5 = _make_gather(bt, f, v, d, 4)(x3, tab_rm)
    return out5.transpose(2, 4, 0, 1, 3).reshape(bt, f, d)


# speedup vs baseline: 1.8755x; 1.8755x over previous
import functools
import math

import jax
import jax.numpy as jnp
from jax import lax
from jax.experimental import pallas as pl
from jax.experimental.pallas import tpu as pltpu
from jax.experimental.pallas import tpu_sc as plsc

_LANE = 16
_TL = 128


def _mesh():
    return plsc.VectorSubcoreMesh(core_axis_name="c", subcore_axis_name="s")


def _wid(nc):
    return lax.axis_index("s") * nc + lax.axis_index("c")


@functools.lru_cache(maxsize=None)
def _make_retile(V, D, S):
    info = plsc.get_sparse_core_info()
    nc, ns = info.num_cores, info.num_subcores
    nw = nc * ns
    n_blk = V // _TL
    rem = V - n_blk * _TL
    n_outer = (n_blk + nw - 1) // nw
    rpb = _TL * D // _TL
    assert D % _LANE == 0 and (rem * D) % _TL == 0

    @functools.partial(
        pl.kernel,
        mesh=_mesh(),
        compiler_params=pltpu.CompilerParams(
            use_tc_tiling_on_sc=True, needs_layout_passes=False
        ),
        out_type=jax.ShapeDtypeStruct((V * D // _TL, _TL), jnp.float32),
        scratch_types=[
            pltpu.VMEM((S, D, _TL), jnp.float32),
            pltpu.VMEM((S, rpb, _TL), jnp.float32),
            pltpu.VMEM((D, rem if rem else 1), jnp.float32),
            pltpu.VMEM((max(rem * D // _TL, 1), _TL), jnp.float32),
        ]
        + [pltpu.SemaphoreType.DMA] * (2 * S),
    )
    def retile(tt_hbm, out_hbm, tin_v, tout_v, rin_v, rout_v, *sems):
        sem_g = sems[:S]
        sem_w = sems[S:]
        w = _wid(nc)
        iota = lax.iota(jnp.int32, _LANE)
        rows01 = [iota + j * _LANE for j in range(D // _LANE)]
        vper = _TL // D

        def fire_read(b, c):
            pltpu.async_copy(
                tt_hbm.at[:, pl.ds(c * _TL, _TL)], tin_v.at[b], sem_g[b]
            )

        def drain_read(b):
            pltpu.make_async_copy(
                tt_hbm.at[:, pl.ds(0, _TL)], tin_v.at[b], sem_g[b]
            ).wait()

        def wait_write(b):
            pltpu.make_async_copy(
                tout_v.at[b], out_hbm.at[pl.ds(0, rpb)], sem_w[b]
            ).wait()

        def transpose_blk(src, dst, nv):
            lpd = D // _LANE

            @plsc.parallel_loop(0, nv * D // _LANE, unroll=8)
            def _(m):
                vp = m // lpd
                row = iota + (m % lpd) * _LANE
                col = jnp.full((_LANE,), 0, jnp.int32) + vp
                vals = plsc.load_gather(src, [row, col])
                flat = m * _LANE
                dst[flat // _TL, pl.ds(flat % _TL, _LANE)] = vals

        for b in range(S):
            c = b * nw + w

            @pl.when(c < n_blk)
            def _():
                fire_read(b, c)

        def outer(g, carry):
            for b in range(S):
                c = (g * S + b) * nw + w

                @pl.when(c < n_blk)
                def _():
                    drain_read(b)
                    transpose_blk(tin_v.at[b], tout_v.at[b], _TL)
                    pltpu.async_copy(
                        tout_v.at[b],
                        out_hbm.at[pl.ds(c * rpb, rpb)],
                        sem_w[b],
                    )
                    t = c + S * nw

                    @pl.when(t < n_blk)
                    def _():
                        wait_write(b)
                        fire_read(b, t)
            return carry

        lax.fori_loop(0, n_outer // S + 1, outer, 0)
        for b in range(S):
            c0 = b * nw + w

            @pl.when(c0 < n_blk)
            def _():
                wait_write(b)

        if rem:
            @pl.when(w == nw - 1)
            def _():
                pltpu.sync_copy(tt_hbm.at[:, pl.ds(n_blk * _TL, rem)], rin_v)
                transpose_blk(rin_v, rout_v, rem)
                pltpu.sync_copy(
                    rout_v,
                    out_hbm.at[pl.ds(n_blk * rpb, rem * D // _TL)],
                )

    return retile


@functools.lru_cache(maxsize=None)
def _make_gather(BT, F, V, D, S):
    info = plsc.get_sparse_core_info()
    nc, ns = info.num_cores, info.num_subcores
    nw = nc * ns
    bblk = BT // _TL
    n_chunk = F * bblk
    per_w = n_chunk // nw
    scale = math.sqrt(D)
    assert BT % _TL == 0 and n_chunk % nw == 0 and per_w % S == 0

    @functools.partial(
        pl.kernel,
        mesh=_mesh(),
        compiler_params=pltpu.CompilerParams(use_tc_tiling_on_sc=False, needs_layout_passes=False),
        out_type=jax.ShapeDtypeStruct((F, D // 8, bblk, 8, _TL), jnp.float32),
        scratch_types=[
            pltpu.VMEM((S, _TL), jnp.int32),
            pltpu.VMEM((S, _TL, D), jnp.float32),
            pltpu.VMEM((S, D // 8, 8, _TL), jnp.float32),
        ]
        + [pltpu.SemaphoreType.DMA] * (2 * S),
    )
    def gatherk(x3_hbm, tab_hbm, out_hbm, ix_v, rows_v, outt_v, *sems):
        sem_g = sems[:S]
        sem_w = sems[S:]
        w = _wid(nc)
        cbase = w * per_w
        iota = lax.iota(jnp.int32, _LANE)
        lanes = [iota + i * _LANE for i in range(_TL // _LANE)]

        def fetch_and_fire(b, c):
            f = c // bblk
            bb = c % bblk
            pltpu.sync_copy(x3_hbm.at[f, bb], ix_v.at[b])
            pltpu.async_copy(tab_hbm.at[ix_v.at[b]], rows_v.at[b], sem_g[b])

        def drain_gather(b):
            pltpu.make_async_copy(
                tab_hbm.at[pl.ds(0, _TL)], rows_v.at[b], sem_g[b]
            ).wait()

        def wait_write(b):
            pltpu.make_async_copy(
                outt_v.at[b], out_hbm.at[0, :, 0], sem_w[b]
            ).wait()

        def extract(b):
            @plsc.parallel_loop(0, (_TL // _LANE) * D, unroll=8)
            def _(m):
                i = m // D
                d = m % D
                row = iota + i * _LANE
                col = jnp.full((_LANE,), 0, jnp.int32) + d
                vals = plsc.load_gather(rows_v.at[b], [row, col])
                outt_v[b, d // 8, d % 8, pl.ds(i * _LANE, _LANE)] = (
                    vals * scale
                )

        for b in range(S):
            fetch_and_fire(b, cbase + b)

        def outer(g, carry):
            for b in range(S):
                c = cbase + g * S + b
                drain_gather(b)
                extract(b)
                f = c // bblk
                bb = c % bblk
                pltpu.async_copy(
                    outt_v.at[b], out_hbm.at[f, :, bb], sem_w[b]
                )
                bp = (b - 1) % S
                t = c + S - 1
                ok = jnp.logical_and(g * S + b >= 1,
                                     t <= cbase + per_w - 1)

                @pl.when(ok)
                def _():
                    wait_write(bp)
                    fetch_and_fire(bp, t)
            return carry

        lax.fori_loop(0, per_w // S, outer, 0)
        for b in range(S):
            wait_write(b)

    return gatherk


def kernel(x, table):
    bt, f = x.shape
    v, d = table.shape
    tab128 = _make_retile(v, d, 3)(table.T)
    tab_rm = tab128.reshape(v, d)
    x3 = x.T.astype(jnp.int32).reshape(f, bt // _TL, _TL)
    out5 = _make_gather(bt, f, v, d, 4)(x3, tab_rm)
    return out5.transpose(2, 4, 0, 1, 3).reshape(bt, f, d)


# speedup vs baseline: 2.0749x; 1.1063x over previous
import functools
import math

import jax
import jax.numpy as jnp
from jax import lax
from jax.experimental import pallas as pl
from jax.experimental.pallas import tpu as pltpu
from jax.experimental.pallas import tpu_sc as plsc

_LANE = 16
_TL = 128


def _mesh():
    return plsc.VectorSubcoreMesh(core_axis_name="c", subcore_axis_name="s")


def _wid(nc):
    return lax.axis_index("s") * nc + lax.axis_index("c")


@functools.lru_cache(maxsize=None)
def _make_retile(V, D, S):
    info = plsc.get_sparse_core_info()
    nc, ns = info.num_cores, info.num_subcores
    nw = nc * ns
    n_blk = V // _TL
    rem = V - n_blk * _TL
    n_outer = (n_blk + nw - 1) // nw
    rpb = _TL * D // _TL
    assert D % _LANE == 0 and (rem * D) % _TL == 0

    @functools.partial(
        pl.kernel,
        mesh=_mesh(),
        compiler_params=pltpu.CompilerParams(
            use_tc_tiling_on_sc=True, needs_layout_passes=False
        ),
        out_type=jax.ShapeDtypeStruct((V * D // _TL, _TL), jnp.float32),
        scratch_types=[
            pltpu.VMEM((S, D, _TL), jnp.float32),
            pltpu.VMEM((S, rpb, _TL), jnp.float32),
            pltpu.VMEM((D, rem if rem else 1), jnp.float32),
            pltpu.VMEM((max(rem * D // _TL, 1), _TL), jnp.float32),
        ]
        + [pltpu.SemaphoreType.DMA] * (2 * S),
    )
    def retile(tt_hbm, out_hbm, tin_v, tout_v, rin_v, rout_v, *sems):
        sem_g = sems[:S]
        sem_w = sems[S:]
        w = _wid(nc)
        iota = lax.iota(jnp.int32, _LANE)
        rows01 = [iota + j * _LANE for j in range(D // _LANE)]
        vper = _TL // D

        def fire_read(b, c):
            pltpu.async_copy(
                tt_hbm.at[:, pl.ds(c * _TL, _TL)], tin_v.at[b], sem_g[b]
            )

        def drain_read(b):
            pltpu.make_async_copy(
                tt_hbm.at[:, pl.ds(0, _TL)], tin_v.at[b], sem_g[b]
            ).wait()

        def wait_write(b):
            pltpu.make_async_copy(
                tout_v.at[b], out_hbm.at[pl.ds(0, rpb)], sem_w[b]
            ).wait()

        def transpose_blk(src, dst, nv):
            @plsc.parallel_loop(0, nv, unroll=8)
            def _(vp):
                col = jnp.full((_LANE,), 0, jnp.int32) + vp
                flat = vp * D
                for j in range(D // _LANE):
                    vals = plsc.load_gather(src, [rows01[j], col])
                    f2 = flat + j * _LANE
                    dst[f2 // _TL, pl.ds(f2 % _TL, _LANE)] = vals

        for b in range(S):
            c = b * nw + w

            @pl.when(c < n_blk)
            def _():
                fire_read(b, c)

        def outer(g, carry):
            for b in range(S):
                c = (g * S + b) * nw + w

                @pl.when(c < n_blk)
                def _():
                    drain_read(b)
                    transpose_blk(tin_v.at[b], tout_v.at[b], _TL)
                    pltpu.async_copy(
                        tout_v.at[b],
                        out_hbm.at[pl.ds(c * rpb, rpb)],
                        sem_w[b],
                    )
                    t = c + S * nw

                    @pl.when(t < n_blk)
                    def _():
                        wait_write(b)
                        fire_read(b, t)
            return carry

        lax.fori_loop(0, n_outer // S + 1, outer, 0)
        for b in range(S):
            c0 = b * nw + w

            @pl.when(c0 < n_blk)
            def _():
                wait_write(b)

        if rem:
            @pl.when(w == nw - 1)
            def _():
                pltpu.sync_copy(tt_hbm.at[:, pl.ds(n_blk * _TL, rem)], rin_v)
                transpose_blk(rin_v, rout_v, rem)
                pltpu.sync_copy(
                    rout_v,
                    out_hbm.at[pl.ds(n_blk * rpb, rem * D // _TL)],
                )

    return retile


@functools.lru_cache(maxsize=None)
def _make_gather(BT, F, V, D, S):
    info = plsc.get_sparse_core_info()
    nc, ns = info.num_cores, info.num_subcores
    nw = nc * ns
    bblk = BT // _TL
    n_chunk = F * bblk
    per_w = n_chunk // nw
    scale = math.sqrt(D)
    assert BT % _TL == 0 and n_chunk % nw == 0 and per_w % S == 0

    @functools.partial(
        pl.kernel,
        mesh=_mesh(),
        compiler_params=pltpu.CompilerParams(use_tc_tiling_on_sc=False, needs_layout_passes=False),
        out_type=jax.ShapeDtypeStruct((F, D // 8, bblk, 8, _TL), jnp.float32),
        scratch_types=[
            pltpu.VMEM((S, _TL), jnp.int32),
            pltpu.VMEM((S, _TL, D), jnp.float32),
            pltpu.VMEM((S, D // 8, 8, _TL), jnp.float32),
        ]
        + [pltpu.SemaphoreType.DMA] * (2 * S),
    )
    def gatherk(x3_hbm, tab_hbm, out_hbm, ix_v, rows_v, outt_v, *sems):
        sem_g = sems[:S]
        sem_w = sems[S:]
        w = _wid(nc)
        cbase = w * per_w
        iota = lax.iota(jnp.int32, _LANE)
        lanes = [iota + i * _LANE for i in range(_TL // _LANE)]

        def fetch_and_fire(b, c):
            f = c // bblk
            bb = c % bblk
            pltpu.sync_copy(x3_hbm.at[f, bb], ix_v.at[b])
            pltpu.async_copy(tab_hbm.at[ix_v.at[b]], rows_v.at[b], sem_g[b])

        def drain_gather(b):
            pltpu.make_async_copy(
                tab_hbm.at[pl.ds(0, _TL)], rows_v.at[b], sem_g[b]
            ).wait()

        def wait_write(b):
            pltpu.make_async_copy(
                outt_v.at[b], out_hbm.at[0, :, 0], sem_w[b]
            ).wait()

        def extract(b):
            @plsc.parallel_loop(0, D, unroll=4)
            def _(d):
                col = jnp.full((_LANE,), 0, jnp.int32) + d
                dg = d // 8
                ds_ = d % 8
                for i in range(_TL // _LANE):
                    vals = plsc.load_gather(rows_v.at[b], [lanes[i], col])
                    outt_v[b, dg, ds_, pl.ds(i * _LANE, _LANE)] = (
                        vals * scale
                    )

        for b in range(S):
            fetch_and_fire(b, cbase + b)

        def outer(g, carry):
            for b in range(S):
                c = cbase + g * S + b
                drain_gather(b)
                extract(b)
                f = c // bblk
                bb = c % bblk
                pltpu.async_copy(
                    outt_v.at[b], out_hbm.at[f, :, bb], sem_w[b]
                )
                bp = (b - 1) % S
                t = c + S - 1
                ok = jnp.logical_and(g * S + b >= 1,
                                     t <= cbase + per_w - 1)

                @pl.when(ok)
                def _():
                    wait_write(bp)
                    fetch_and_fire(bp, t)
            return carry

        lax.fori_loop(0, per_w // S, outer, 0)
        for b in range(S):
            wait_write(b)

    return gatherk


def kernel(x, table):
    bt, f = x.shape
    v, d = table.shape
    tab128 = _make_retile(v, d, 3)(table.T)
    tab_rm = tab128.reshape(v, d)
    x3 = x.T.astype(jnp.int32).reshape(f, bt // _TL, _TL)
    out5 = _make_gather(bt, f, v, d, 4)(x3, tab_rm)
    return out5.transpose(2, 4, 0, 1, 3).reshape(bt, f, d)


# speedup vs baseline: 2.4226x; 1.1676x over previous
import functools
import math

import jax
import jax.numpy as jnp
from jax import lax
from jax.experimental import pallas as pl
from jax.experimental.pallas import tpu as pltpu
from jax.experimental.pallas import tpu_sc as plsc

_LANE = 16
_TL = 128


def _mesh():
    return plsc.VectorSubcoreMesh(core_axis_name="c", subcore_axis_name="s")


def _wid(nc):
    return lax.axis_index("s") * nc + lax.axis_index("c")


@functools.lru_cache(maxsize=None)
def _make_retile(V, D, S):
    info = plsc.get_sparse_core_info()
    nc, ns = info.num_cores, info.num_subcores
    nw = nc * ns
    n_blk = V // _TL
    rem = V - n_blk * _TL
    n_outer = (n_blk + nw - 1) // nw
    rpb = _TL * D // _TL
    assert D % _LANE == 0 and (rem * D) % _TL == 0

    @functools.partial(
        pl.kernel,
        mesh=_mesh(),
        compiler_params=pltpu.CompilerParams(
            use_tc_tiling_on_sc=True, needs_layout_passes=False
        ),
        out_type=jax.ShapeDtypeStruct((V * D // _TL, _TL), jnp.float32),
        scratch_types=[
            pltpu.VMEM((S, D, _TL), jnp.float32),
            pltpu.VMEM((S, rpb, _TL), jnp.float32),
            pltpu.VMEM((D, rem if rem else 1), jnp.float32),
            pltpu.VMEM((max(rem * D // _TL, 1), _TL), jnp.float32),
        ]
        + [pltpu.SemaphoreType.DMA] * (2 * S),
    )
    def retile(tt_hbm, out_hbm, tin_v, tout_v, rin_v, rout_v, *sems):
        sem_g = sems[:S]
        sem_w = sems[S:]
        w = _wid(nc)
        iota = lax.iota(jnp.int32, _LANE)
        rows01 = [iota + j * _LANE for j in range(D // _LANE)]
        vper = _TL // D

        def fire_read(b, c):
            pltpu.async_copy(
                tt_hbm.at[:, pl.ds(c * _TL, _TL)], tin_v.at[b], sem_g[b]
            )

        def drain_read(b):
            pltpu.make_async_copy(
                tt_hbm.at[:, pl.ds(0, _TL)], tin_v.at[b], sem_g[b]
            ).wait()

        def wait_write(b):
            pltpu.make_async_copy(
                tout_v.at[b], out_hbm.at[pl.ds(0, rpb)], sem_w[b]
            ).wait()

        def transpose_blk(src, dst, nv):
            @plsc.parallel_loop(0, nv, unroll=8)
            def _(vp):
                col = jnp.full((_LANE,), 0, jnp.int32) + vp
                flat = vp * D
                for j in range(D // _LANE):
                    vals = plsc.load_gather(src, [rows01[j], col])
                    f2 = flat + j * _LANE
                    dst[f2 // _TL, pl.ds(f2 % _TL, _LANE)] = vals

        for b in range(S):
            c = b * nw + w

            @pl.when(c < n_blk)
            def _():
                fire_read(b, c)

        def outer(g, carry):
            for b in range(S):
                j = g * S + b
                c = j * nw + w

                @pl.when(c < n_blk)
                def _():
                    drain_read(b)

                    @pl.when(j >= S)
                    def _():
                        wait_write(b)

                    transpose_blk(tin_v.at[b], tout_v.at[b], _TL)
                    pltpu.async_copy(
                        tout_v.at[b],
                        out_hbm.at[pl.ds(c * rpb, rpb)],
                        sem_w[b],
                    )
                    t = c + S * nw

                    @pl.when(t < n_blk)
                    def _():
                        fire_read(b, t)
            return carry

        lax.fori_loop(0, n_outer // S + 1, outer, 0)
        for b in range(S):
            c0 = b * nw + w

            @pl.when(c0 < n_blk)
            def _():
                wait_write(b)

        if rem:
            @pl.when(w == nw - 1)
            def _():
                pltpu.sync_copy(tt_hbm.at[:, pl.ds(n_blk * _TL, rem)], rin_v)
                transpose_blk(rin_v, rout_v, rem)
                pltpu.sync_copy(
                    rout_v,
                    out_hbm.at[pl.ds(n_blk * rpb, rem * D // _TL)],
                )

    return retile


@functools.lru_cache(maxsize=None)
def _make_gather(BT, F, V, D, S):
    info = plsc.get_sparse_core_info()
    nc, ns = info.num_cores, info.num_subcores
    nw = nc * ns
    bblk = BT // _TL
    n_chunk = F * bblk
    per_w = n_chunk // nw
    scale = math.sqrt(D)
    assert BT % _TL == 0 and n_chunk % nw == 0 and per_w % S == 0

    @functools.partial(
        pl.kernel,
        mesh=_mesh(),
        compiler_params=pltpu.CompilerParams(use_tc_tiling_on_sc=False, needs_layout_passes=False),
        out_type=jax.ShapeDtypeStruct((F, D // 8, bblk, 8, _TL), jnp.float32),
        scratch_types=[
            pltpu.VMEM((S, _TL), jnp.int32),
            pltpu.VMEM((S, _TL, D), jnp.float32),
            pltpu.VMEM((S, D // 8, 8, _TL), jnp.float32),
        ]
        + [pltpu.SemaphoreType.DMA] * (3 * S),
    )
    def gatherk(x3_hbm, tab_hbm, out_hbm, ix_v, rows_v, outt_v, *sems):
        sem_g = sems[:S]
        sem_w = sems[S:2 * S]
        sem_i = sems[2 * S:]
        w = _wid(nc)
        cbase = w * per_w
        iota = lax.iota(jnp.int32, _LANE)
        lanes = [iota + i * _LANE for i in range(_TL // _LANE)]

        def fire_ix(b, c):
            f = c // bblk
            bb = c % bblk
            pltpu.async_copy(x3_hbm.at[f, bb], ix_v.at[b], sem_i[b])

        def wait_ix(b):
            pltpu.make_async_copy(
                x3_hbm.at[0, 0], ix_v.at[b], sem_i[b]
            ).wait()

        def fire_gather(b):
            pltpu.async_copy(tab_hbm.at[ix_v.at[b]], rows_v.at[b], sem_g[b])

        def drain_gather(b):
            pltpu.make_async_copy(
                tab_hbm.at[pl.ds(0, _TL)], rows_v.at[b], sem_g[b]
            ).wait()

        def wait_write(b):
            pltpu.make_async_copy(
                outt_v.at[b], out_hbm.at[0, :, 0], sem_w[b]
            ).wait()

        def extract(b):
            @plsc.parallel_loop(0, D, unroll=4)
            def _(d):
                col = jnp.full((_LANE,), 0, jnp.int32) + d
                dg = d // 8
                ds_ = d % 8
                for i in range(_TL // _LANE):
                    vals = plsc.load_gather(rows_v.at[b], [lanes[i], col])
                    outt_v[b, dg, ds_, pl.ds(i * _LANE, _LANE)] = (
                        vals * scale
                    )

        for b in range(S):
            fire_ix(b, cbase + b)
            wait_ix(b)
            fire_gather(b)

        def outer(g, carry):
            for b in range(S):
                c = cbase + g * S + b
                t = c + S
                ok = t <= cbase + per_w - 1
                drain_gather(b)

                @pl.when(ok)
                def _():
                    fire_ix(b, t)

                @pl.when(g >= 1)
                def _():
                    wait_write(b)

                extract(b)
                f = c // bblk
                bb = c % bblk
                pltpu.async_copy(
                    outt_v.at[b], out_hbm.at[f, :, bb], sem_w[b]
                )

                @pl.when(ok)
                def _():
                    wait_ix(b)
                    fire_gather(b)
            return carry

        lax.fori_loop(0, per_w // S, outer, 0)
        for b in range(S):
            wait_write(b)

    return gatherk


def kernel(x, table):
    bt, f = x.shape
    v, d = table.shape
    tab128 = _make_retile(v, d, 3)(table.T)
    tab_rm = tab128.reshape(v, d)
    x3 = x.T.astype(jnp.int32).reshape(f, bt // _TL, _TL)
    out5 = _make_gather(bt, f, v, d, 4)(x3, tab_rm)
    return out5.transpose(2, 4, 0, 1, 3).reshape(bt, f, d)
